# baseline (speedup 1.0000x reference)
import functools

import jax
import jax.numpy as jnp
from jax import lax
from jax.experimental import pallas as pl
from jax.experimental.pallas import tpu as pltpu
from jax.experimental.pallas import tpu_sc as plsc

_N = 100000
_E = 1600000
_IN = 11
_H = 64
_G = 64

_NB = 2048
_GRID = 49
_NPAD = _NB * _GRID
_EPAD = 1638400
_ROWS = _EPAD // 128
_CH = 8
_M1 = 50
_M2 = 100
_TS = _NPAD // 16

@functools.cache
def _sc_kernels():
    mesh = plsc.VectorSubcoreMesh(
        core_axis_name="c", subcore_axis_name="s", num_cores=2, num_subcores=16)
    params = pltpu.CompilerParams(use_tc_tiling_on_sc=False)
    deg = functools.partial(
        pl.kernel,
        out_type=jax.ShapeDtypeStruct((2, _NPAD), jnp.float32),
        mesh=mesh,
        scratch_types=[
            pltpu.VMEM((_CH, 128), jnp.int32),
            pltpu.VMEM((2, 4, 128), jnp.int32),
            pltpu.VMEM((128,), jnp.float32),
            pltpu.VMEM_SHARED((_NPAD,), jnp.float32),
            pltpu.SemaphoreType.DMA,
            pltpu.SemaphoreType.DMA,
        ],
        compiler_params=params,
    )(_deg_body)
    scat1 = functools.partial(
        pl.kernel,
        out_type=jax.ShapeDtypeStruct((2, _NPAD, 16), jnp.float32),
        mesh=mesh,
        scratch_types=[
            pltpu.VMEM((_CH, 128), jnp.int32),
            pltpu.VMEM((_CH, 128), jnp.int32),
            pltpu.VMEM((2, 4, 128), jnp.int32),
            pltpu.VMEM((2, 4, 128), jnp.int32),
            pltpu.VMEM((2, 4, 128, 16), jnp.float32),
            pltpu.VMEM_SHARED((_NPAD, 16), jnp.float32),
            pltpu.SemaphoreType.DMA,
            pltpu.SemaphoreType.DMA,
            pltpu.SemaphoreType.DMA,
        ],
        compiler_params=params,
    )(_scat1_body)
    scat2 = functools.partial(
        pl.kernel,
        out_type=jax.ShapeDtypeStruct((4, _NPAD, 16), jnp.float32),
        mesh=mesh,
        scratch_types=[
            pltpu.VMEM((_CH, 128), jnp.int32),
            pltpu.VMEM((_CH, 128), jnp.int32),
            pltpu.VMEM((2, 4, 128), jnp.int32),
            pltpu.VMEM((2, 4, 128), jnp.int32),
            pltpu.VMEM((2, 4, 128, 16), jnp.float32),
            pltpu.VMEM_SHARED((_NPAD, 16), jnp.float32),
            pltpu.SemaphoreType.DMA,
            pltpu.SemaphoreType.DMA,
            pltpu.SemaphoreType.DMA,
        ],
        compiler_params=params,
    )(_scat2_body)
    return deg, scat1, scat2


def _pipe_pass(src_hbm, dst_hbm, tab_hbm, acc, ibs, ibd, gb, db, rows,
               semI, semG, semS, base_blk, M, gidx_fn):
    def load_idx(m):
        blk = base_blk + jnp.minimum(m, M - 1)
        pltpu.async_copy(src_hbm.at[blk], ibs, semI)
        pltpu.async_copy(dst_hbm.at[blk], ibd, semI)

    def wait_idx():
        pltpu.make_async_copy(src_hbm.at[0], ibs, semI).wait()
        pltpu.make_async_copy(dst_hbm.at[0], ibd, semI).wait()

    def compute(h, k):
        for j in range(4):
            for o in range(8):
                sl = pl.ds(o * 16, 16)
                gb[k, j, sl] = gidx_fn(ibs[4 * h + j, sl])
                db[k, j, sl] = ibd[4 * h + j, sl]

    def fire_g(k):
        for j in range(4):
            pltpu.async_copy(tab_hbm.at[gb.at[k, j]], rows.at[k, j], semG)

    def wait_g(k):
        for j in range(4):
            pltpu.make_async_copy(tab_hbm.at[pl.ds(0, 128)], rows.at[k, j],
                                  semG).wait()

    def fire_s(k):
        for j in range(4):
            pltpu.async_copy(rows.at[k, j], acc.at[db.at[k, j]], semS, add=True)

    def wait_s(k):
        for j in range(4):
            pltpu.make_async_copy(tab_hbm.at[pl.ds(0, 128)], rows.at[k, j],
                                  semS).wait()

    load_idx(0)
    wait_idx()
    compute(0, 0)
    fire_g(0)
    wait_g(0)
    fire_s(0)
    compute(1, 1)
    fire_g(1)
    load_idx(1)

    @pl.loop(0, M - 1)
    def _pairs(p):
        wait_g(1)
        fire_s(1)
        wait_s(0)
        wait_idx()
        compute(0, 0)
        fire_g(0)
        wait_g(0)
        fire_s(0)
        wait_s(1)
        compute(1, 1)
        fire_g(1)
        load_idx(p + 2)

    wait_g(1)
    fire_s(1)
    wait_s(0)
    wait_s(1)
    wait_idx()


def _deg_body(dst_hbm, zf_hbm, out_hbm, ibd, db, ones_v, acc, semI, semS):
    core = lax.axis_index("c")
    sub = lax.axis_index("s")
    wid = sub * 2 + core
    for o in range(8):
        ones_v[pl.ds(o * 16, 16)] = jnp.ones((16,), jnp.float32)
    pltpu.sync_copy(zf_hbm.at[pl.ds(sub * _TS, _TS)], acc.at[pl.ds(sub * _TS, _TS)])
    plsc.subcore_barrier()
    base_blk = wid * _M1

    def load_idx(m):
        blk = base_blk + jnp.minimum(m, _M1 - 1)
        pltpu.async_copy(dst_hbm.at[blk], ibd, semI)

    def wait_idx():
        pltpu.make_async_copy(dst_hbm.at[0], ibd, semI).wait()

    def compute(h, k):
        for j in range(4):
            for o in range(8):
                sl = pl.ds(o * 16, 16)
                db[k, j, sl] = ibd[4 * h + j, sl]

    def fire_s(k):
        for j in range(4):
            pltpu.async_copy(ones_v, acc.at[db.at[k, j]], semS, add=True)

    def wait_s(k):
        for j in range(4):
            pltpu.make_async_copy(zf_hbm.at[pl.ds(0, 128)], ones_v, semS).wait()

    load_idx(0)
    wait_idx()
    compute(0, 0)
    fire_s(0)
    compute(1, 1)
    fire_s(1)
    load_idx(1)

    @pl.loop(0, _M1 - 1)
    def _pairs(p):
        wait_s(0)
        wait_idx()
        compute(0, 0)
        fire_s(0)
        wait_s(1)
        compute(1, 1)
        fire_s(1)
        load_idx(p + 2)

    wait_s(0)
    wait_s(1)
    wait_idx()
    plsc.subcore_barrier()
    pltpu.sync_copy(acc.at[pl.ds(sub * _TS, _TS)],
                    out_hbm.at[core, pl.ds(sub * _TS, _TS)])


def _scat1_body(src_hbm, dst_hbm, tab_hbm, z16_hbm, out_hbm,
                ibs, ibd, gb, db, rows, acc, semI, semG, semS):
    core = lax.axis_index("c")
    sub = lax.axis_index("s")
    wid = sub * 2 + core
    pltpu.sync_copy(z16_hbm.at[pl.ds(sub * _TS, _TS)], acc.at[pl.ds(sub * _TS, _TS)])
    plsc.subcore_barrier()
    _pipe_pass(src_hbm, dst_hbm, tab_hbm, acc, ibs, ibd, gb, db, rows,
               semI, semG, semS, wid * _M1, _M1, lambda v: v)
    plsc.subcore_barrier()
    pltpu.sync_copy(acc.at[pl.ds(sub * _TS, _TS)],
                    out_hbm.at[core, pl.ds(sub * _TS, _TS)])


def _scat2_body(src_hbm, dst_hbm, tab_hbm, z16_hbm, out_hbm,
                ibs, ibd, gb, db, rows, acc, semI, semG, semS):
    core = lax.axis_index("c")
    sub = lax.axis_index("s")
    for cc in range(2):
        c = core * 2 + cc
        pltpu.sync_copy(z16_hbm.at[pl.ds(sub * _TS, _TS)],
                        acc.at[pl.ds(sub * _TS, _TS)])
        plsc.subcore_barrier()
        _pipe_pass(src_hbm, dst_hbm, tab_hbm, acc, ibs, ibd, gb, db, rows,
                   semI, semG, semS, sub * _M2, _M2, lambda v: v * 4 + c)
        plsc.subcore_barrier()
        pltpu.sync_copy(acc.at[pl.ds(sub * _TS, _TS)],
                        out_hbm.at[c, pl.ds(sub * _TS, _TS)])


def _prep_body(indeg_ref, x_ref, dinv_ref, xs_ref):
    i = pl.program_id(0)
    ind = indeg_ref[...]
    s = ind[0] + ind[1]
    row = lax.broadcasted_iota(jnp.int32, (_NB, 1), 0) + i * _NB
    dinv = jnp.where(row < _N, lax.rsqrt(s + 1.0), 0.0)
    dinv_ref[...] = dinv
    xs_ref[...] = x_ref[...] * dinv


_prep_tc = pl.pallas_call(
    _prep_body,
    grid=(_GRID,),
    in_specs=[
        pl.BlockSpec((2, _NB, 1), lambda i: (0, i, 0)),
        pl.BlockSpec((_NB, 16), lambda i: (i, 0)),
    ],
    out_specs=[
        pl.BlockSpec((_NB, 1), lambda i: (i, 0)),
        pl.BlockSpec((_NB, 16), lambda i: (i, 0)),
    ],
    out_shape=[
        jax.ShapeDtypeStruct((_NPAD, 1), jnp.float32),
        jax.ShapeDtypeStruct((_NPAD, 16), jnp.float32),
    ],
)


def _l1_body(acc_ref, xs_ref, dinv_ref, w_ref, b_ref, tab_ref):
    a = acc_ref[...]
    t = a[0] + a[1] + xs_ref[...]
    h = jnp.dot(t, w_ref[...], preferred_element_type=jnp.float32)
    dinv = dinv_ref[...]
    out1 = jnp.maximum(h * dinv + b_ref[...], 0.0)
    tab_ref[...] = out1 * dinv


_l1_tc = pl.pallas_call(
    _l1_body,
    grid=(_GRID,),
    in_specs=[
        pl.BlockSpec((2, _NB, 16), lambda i: (0, i, 0)),
        pl.BlockSpec((_NB, 16), lambda i: (i, 0)),
        pl.BlockSpec((_NB, 1), lambda i: (i, 0)),
        pl.BlockSpec((16, _H), lambda i: (0, 0)),
        pl.BlockSpec((1, _H), lambda i: (0, 0)),
    ],
    out_specs=pl.BlockSpec((_NB, _H), lambda i: (i, 0)),
    out_shape=jax.ShapeDtypeStruct((_NPAD, _H), jnp.float32),
)


def _l2_body(acc_ref, tab_ref, dinv_ref, batch_ref, w2_ref, b2_ref,
             lw1_ref, lb1_ref, lw2_ref, lb2_ref, out_ref, sums, cnts):
    i = pl.program_id(0)

    @pl.when(i == 0)
    def _():
        sums[...] = jnp.zeros((_G, _H), jnp.float32)
        cnts[...] = jnp.zeros((_G, 1), jnp.float32)

    a = acc_ref[...]
    acc = jnp.concatenate([a[0], a[1], a[2], a[3]], axis=-1)
    t = acc + tab_ref[...]
    h = jnp.dot(t, w2_ref[...], preferred_element_type=jnp.float32)
    h2 = jnp.maximum(h * dinv_ref[...] + b2_ref[...], 0.0)
    b = batch_ref[0]
    io = lax.broadcasted_iota(jnp.int32, (_G, _NB), 0)
    oh = jnp.where(io == b, 1.0, 0.0)
    sums[...] += jnp.dot(oh, h2, preferred_element_type=jnp.float32)
    cnts[...] += jnp.sum(oh, axis=1, keepdims=True)

    @pl.when(i == _GRID - 1)
    def _():
        p = sums[...] / jnp.maximum(cnts[...], 1.0)
        q = jnp.maximum(
            jnp.dot(p, lw1_ref[...], preferred_element_type=jnp.float32)
            + lb1_ref[...], 0.0)
        out_ref[...] = (jnp.dot(q, lw2_ref[...], preferred_element_type=jnp.float32)
                        + lb2_ref[...])


_l2_tc = pl.pallas_call(
    _l2_body,
    grid=(_GRID,),
    in_specs=[
        pl.BlockSpec((4, _NB, 16), lambda i: (0, i, 0)),
        pl.BlockSpec((_NB, _H), lambda i: (i, 0)),
        pl.BlockSpec((_NB, 1), lambda i: (i, 0)),
        pl.BlockSpec((1, 1, _NB), lambda i: (i, 0, 0)),
        pl.BlockSpec((_H, _H), lambda i: (0, 0)),
        pl.BlockSpec((1, _H), lambda i: (0, 0)),
        pl.BlockSpec((_H, _H), lambda i: (0, 0)),
        pl.BlockSpec((1, _H), lambda i: (0, 0)),
        pl.BlockSpec((_H, 1), lambda i: (0, 0)),
        pl.BlockSpec((1, 1), lambda i: (0, 0)),
    ],
    out_specs=pl.BlockSpec((_G, 1), lambda i: (0, 0)),
    out_shape=jax.ShapeDtypeStruct((_G, 1), jnp.float32),
    scratch_shapes=[
        pltpu.VMEM((_G, _H), jnp.float32),
        pltpu.VMEM((_G, 1), jnp.float32),
    ],
)


def kernel(x, edge_index, batch, W1, b1, W2, b2, LW1, Lb1, LW2, Lb2):
    src = edge_index[0]
    dst = edge_index[1]
    pad_e = _EPAD - _E
    src_p = jnp.concatenate([src, jnp.zeros((pad_e,), jnp.int32)])
    dump = _N + (jnp.arange(pad_e, dtype=jnp.int32) % (_NPAD - _N))
    dst_p = jnp.concatenate([dst, dump])
    src2d = src_p.reshape(_ROWS // _CH, _CH, 128)
    dst2d = dst_p.reshape(_ROWS // _CH, _CH, 128)
    zf = jnp.zeros((_NPAD,), jnp.float32)
    z16 = jnp.zeros((_NPAD, 16), jnp.float32)
    xpad = jnp.pad(x, ((0, _NPAD - _N), (0, 16 - _IN)))
    w1p = jnp.pad(W1, ((0, 16 - _IN), (0, 0)))

    deg_sc, scat1_sc, scat2_sc = _sc_kernels()
    indeg = deg_sc(dst2d, zf)
    dinv, xs = _prep_tc(indeg.reshape(2, _NPAD, 1), xpad)
    acc1 = scat1_sc(src2d, dst2d, xs, z16)
    tab2 = _l1_tc(acc1, xs, dinv, w1p, b1.reshape(1, _H))
    acc2 = scat2_sc(src2d, dst2d, tab2.reshape(4 * _NPAD, 16), z16)
    batch3 = jnp.pad(batch, (0, _NPAD - _N), constant_values=_G).reshape(
        _GRID, 1, _NB)
    out = _l2_tc(acc2, tab2, dinv, batch3, W2, b2.reshape(1, _H),
                 LW1, Lb1.reshape(1, _H), LW2, Lb2.reshape(1, 1))
    return out


# speedup vs baseline: 1.4412x; 1.4412x over previous
import functools

import jax
import jax.numpy as jnp
from jax import lax
from jax.experimental import pallas as pl
from jax.experimental.pallas import tpu as pltpu
from jax.experimental.pallas import tpu_sc as plsc

_N = 100000
_E = 1600000
_IN = 11
_H = 64
_G = 64

_NB = 2048
_GRID = 49
_NPAD = _NB * _GRID
_EPAD = 1605632
_ROWS = _EPAD // 128
_CH = 8
_M1 = 49
_M2 = 98
_TS = _NPAD // 16

@functools.cache
def _sc_kernels():
    mesh = plsc.VectorSubcoreMesh(
        core_axis_name="c", subcore_axis_name="s", num_cores=2, num_subcores=16)
    params = pltpu.CompilerParams(use_tc_tiling_on_sc=False)
    deg = functools.partial(
        pl.kernel,
        out_type=jax.ShapeDtypeStruct((2, _NPAD), jnp.float32),
        mesh=mesh,
        scratch_types=[
            pltpu.VMEM((_CH, 128), jnp.int32),
            pltpu.VMEM((2, 4, 128), jnp.int32),
            pltpu.VMEM((128,), jnp.float32),
            pltpu.VMEM_SHARED((_NPAD,), jnp.float32),
            pltpu.SemaphoreType.DMA,
            pltpu.SemaphoreType.DMA,
        ],
        compiler_params=params,
    )(_deg_body)
    scat1 = functools.partial(
        pl.kernel,
        out_type=jax.ShapeDtypeStruct((2, _NPAD, 16), jnp.float32),
        mesh=mesh,
        scratch_types=[
            pltpu.VMEM((_CH, 128), jnp.int32),
            pltpu.VMEM((_CH, 128), jnp.int32),
            pltpu.VMEM((_CH, 128), jnp.int32),
            pltpu.VMEM((_CH, 128), jnp.int32),
            pltpu.VMEM((_CH, 128, 16), jnp.float32),
            pltpu.VMEM_SHARED((_NPAD, 16), jnp.float32),
            pltpu.SemaphoreType.DMA,
            pltpu.SemaphoreType.DMA,
            pltpu.SemaphoreType.DMA,
        ],
        compiler_params=params,
    )(_scat1_body)
    scat2 = functools.partial(
        pl.kernel,
        out_type=jax.ShapeDtypeStruct((4, _NPAD, 16), jnp.float32),
        mesh=mesh,
        scratch_types=[
            pltpu.VMEM((_CH, 128), jnp.int32),
            pltpu.VMEM((_CH, 128), jnp.int32),
            pltpu.VMEM((_CH, 128), jnp.int32),
            pltpu.VMEM((_CH, 128), jnp.int32),
            pltpu.VMEM((_CH, 128, 16), jnp.float32),
            pltpu.VMEM_SHARED((_NPAD, 16), jnp.float32),
            pltpu.SemaphoreType.DMA,
            pltpu.SemaphoreType.DMA,
            pltpu.SemaphoreType.DMA,
        ],
        compiler_params=params,
    )(_scat2_body)
    return deg, scat1, scat2


def _pipe_pass(src_hbm, dst_hbm, tab_hbm, acc, ibs, ibd, gb, db, rows,
               semI, semG, semS, base_blk, M, gidx_fn):
    def load_idx(m):
        blk = base_blk + jnp.minimum(m, M - 1)
        pltpu.async_copy(src_hbm.at[blk], ibs, semI)
        pltpu.async_copy(dst_hbm.at[blk], ibd, semI)

    def wait_idx():
        pltpu.make_async_copy(src_hbm.at[0], ibs, semI).wait()
        pltpu.make_async_copy(dst_hbm.at[0], ibd, semI).wait()

    def compute():
        for j in range(_CH):
            for o in range(8):
                sl = pl.ds(o * 16, 16)
                gb[j, sl] = gidx_fn(ibs[j, sl])
                db[j, sl] = ibd[j, sl]

    def fire_g():
        for j in range(_CH):
            pltpu.async_copy(tab_hbm.at[gb.at[j]], rows.at[j], semG)

    def wait_g():
        for j in range(_CH):
            pltpu.make_async_copy(tab_hbm.at[pl.ds(0, 128)], rows.at[j],
                                  semG).wait()

    def fire_s():
        for j in range(_CH):
            pltpu.async_copy(rows.at[j], acc.at[db.at[j]], semS, add=True)

    def wait_s():
        for j in range(_CH):
            pltpu.make_async_copy(tab_hbm.at[pl.ds(0, 128)], rows.at[j],
                                  semS).wait()

    load_idx(0)
    wait_idx()
    compute()
    load_idx(1)
    fire_g()
    wait_g()
    fire_s()

    @pl.loop(1, M)
    def _steps(m):
        wait_idx()
        wait_s()
        compute()
        load_idx(m + 1)
        fire_g()
        wait_g()
        fire_s()

    wait_s()
    wait_idx()


def _deg_body(dst_hbm, zf_hbm, out_hbm, ibd, db, ones_v, acc, semI, semS):
    core = lax.axis_index("c")
    sub = lax.axis_index("s")
    wid = sub * 2 + core
    for o in range(8):
        ones_v[pl.ds(o * 16, 16)] = jnp.ones((16,), jnp.float32)
    pltpu.sync_copy(zf_hbm.at[pl.ds(sub * _TS, _TS)], acc.at[pl.ds(sub * _TS, _TS)])
    plsc.subcore_barrier()
    base_blk = wid * _M1

    def load_idx(m):
        blk = base_blk + jnp.minimum(m, _M1 - 1)
        pltpu.async_copy(dst_hbm.at[blk], ibd, semI)

    def wait_idx():
        pltpu.make_async_copy(dst_hbm.at[0], ibd, semI).wait()

    def compute(h, k):
        for j in range(4):
            for o in range(8):
                sl = pl.ds(o * 16, 16)
                db[k, j, sl] = ibd[4 * h + j, sl]

    def fire_s(k):
        for j in range(4):
            pltpu.async_copy(ones_v, acc.at[db.at[k, j]], semS, add=True)

    def wait_s(k):
        for j in range(4):
            pltpu.make_async_copy(zf_hbm.at[pl.ds(0, 128)], ones_v, semS).wait()

    load_idx(0)
    wait_idx()
    compute(0, 0)
    fire_s(0)
    compute(1, 1)
    fire_s(1)
    load_idx(1)

    @pl.loop(0, _M1 - 1)
    def _pairs(p):
        wait_s(0)
        wait_idx()
        compute(0, 0)
        fire_s(0)
        wait_s(1)
        compute(1, 1)
        fire_s(1)
        load_idx(p + 2)

    wait_s(0)
    wait_s(1)
    wait_idx()
    plsc.subcore_barrier()
    pltpu.sync_copy(acc.at[pl.ds(sub * _TS, _TS)],
                    out_hbm.at[core, pl.ds(sub * _TS, _TS)])


def _scat1_body(src_hbm, dst_hbm, tab_hbm, z16_hbm, out_hbm,
                ibs, ibd, gb, db, rows, acc, semI, semG, semS):
    core = lax.axis_index("c")
    sub = lax.axis_index("s")
    wid = sub * 2 + core
    pltpu.sync_copy(z16_hbm.at[pl.ds(sub * _TS, _TS)], acc.at[pl.ds(sub * _TS, _TS)])
    plsc.subcore_barrier()
    _pipe_pass(src_hbm, dst_hbm, tab_hbm, acc, ibs, ibd, gb, db, rows,
               semI, semG, semS, wid * _M1, _M1, lambda v: v)
    plsc.subcore_barrier()
    pltpu.sync_copy(acc.at[pl.ds(sub * _TS, _TS)],
                    out_hbm.at[core, pl.ds(sub * _TS, _TS)])


def _scat2_body(src_hbm, dst_hbm, tab_hbm, z16_hbm, out_hbm,
                ibs, ibd, gb, db, rows, acc, semI, semG, semS):
    core = lax.axis_index("c")
    sub = lax.axis_index("s")
    for cc in range(2):
        c = core * 2 + cc
        pltpu.sync_copy(z16_hbm.at[pl.ds(sub * _TS, _TS)],
                        acc.at[pl.ds(sub * _TS, _TS)])
        plsc.subcore_barrier()
        _pipe_pass(src_hbm, dst_hbm, tab_hbm, acc, ibs, ibd, gb, db, rows,
                   semI, semG, semS, sub * _M2, _M2, lambda v: v * 4 + c)
        plsc.subcore_barrier()
        pltpu.sync_copy(acc.at[pl.ds(sub * _TS, _TS)],
                        out_hbm.at[c, pl.ds(sub * _TS, _TS)])


def _prep_body(indeg_ref, x_ref, dinv_ref, xs_ref):
    i = pl.program_id(0)
    ind = indeg_ref[...]
    s = ind[0] + ind[1]
    row = lax.broadcasted_iota(jnp.int32, (_NB, 1), 0) + i * _NB
    dinv = jnp.where(row < _N, lax.rsqrt(s + 1.0), 0.0)
    dinv_ref[...] = dinv
    xs_ref[...] = x_ref[...] * dinv


_prep_tc = pl.pallas_call(
    _prep_body,
    grid=(_GRID,),
    in_specs=[
        pl.BlockSpec((2, _NB, 1), lambda i: (0, i, 0)),
        pl.BlockSpec((_NB, 16), lambda i: (i, 0)),
    ],
    out_specs=[
        pl.BlockSpec((_NB, 1), lambda i: (i, 0)),
        pl.BlockSpec((_NB, 16), lambda i: (i, 0)),
    ],
    out_shape=[
        jax.ShapeDtypeStruct((_NPAD, 1), jnp.float32),
        jax.ShapeDtypeStruct((_NPAD, 16), jnp.float32),
    ],
)


def _l1_body(acc_ref, xs_ref, dinv_ref, w_ref, b_ref, tab_ref):
    a = acc_ref[...]
    t = a[0] + a[1] + xs_ref[...]
    h = jnp.dot(t, w_ref[...], preferred_element_type=jnp.float32)
    dinv = dinv_ref[...]
    out1 = jnp.maximum(h * dinv + b_ref[...], 0.0)
    tab_ref[...] = out1 * dinv


_l1_tc = pl.pallas_call(
    _l1_body,
    grid=(_GRID,),
    in_specs=[
        pl.BlockSpec((2, _NB, 16), lambda i: (0, i, 0)),
        pl.BlockSpec((_NB, 16), lambda i: (i, 0)),
        pl.BlockSpec((_NB, 1), lambda i: (i, 0)),
        pl.BlockSpec((16, _H), lambda i: (0, 0)),
        pl.BlockSpec((1, _H), lambda i: (0, 0)),
    ],
    out_specs=pl.BlockSpec((_NB, _H), lambda i: (i, 0)),
    out_shape=jax.ShapeDtypeStruct((_NPAD, _H), jnp.float32),
)


def _l2_body(acc_ref, tab_ref, dinv_ref, batch_ref, w2_ref, b2_ref,
             lw1_ref, lb1_ref, lw2_ref, lb2_ref, out_ref, sums, cnts):
    i = pl.program_id(0)

    @pl.when(i == 0)
    def _():
        sums[...] = jnp.zeros((_G, _H), jnp.float32)
        cnts[...] = jnp.zeros((_G, 1), jnp.float32)

    a = acc_ref[...]
    acc = jnp.concatenate([a[0], a[1], a[2], a[3]], axis=-1)
    t = acc + tab_ref[...]
    h = jnp.dot(t, w2_ref[...], preferred_element_type=jnp.float32)
    h2 = jnp.maximum(h * dinv_ref[...] + b2_ref[...], 0.0)
    b = batch_ref[0]
    io = lax.broadcasted_iota(jnp.int32, (_G, _NB), 0)
    oh = jnp.where(io == b, 1.0, 0.0)
    sums[...] += jnp.dot(oh, h2, preferred_element_type=jnp.float32)
    cnts[...] += jnp.sum(oh, axis=1, keepdims=True)

    @pl.when(i == _GRID - 1)
    def _():
        p = sums[...] / jnp.maximum(cnts[...], 1.0)
        q = jnp.maximum(
            jnp.dot(p, lw1_ref[...], preferred_element_type=jnp.float32)
            + lb1_ref[...], 0.0)
        out_ref[...] = (jnp.dot(q, lw2_ref[...], preferred_element_type=jnp.float32)
                        + lb2_ref[...])


_l2_tc = pl.pallas_call(
    _l2_body,
    grid=(_GRID,),
    in_specs=[
        pl.BlockSpec((4, _NB, 16), lambda i: (0, i, 0)),
        pl.BlockSpec((_NB, _H), lambda i: (i, 0)),
        pl.BlockSpec((_NB, 1), lambda i: (i, 0)),
        pl.BlockSpec((1, 1, _NB), lambda i: (i, 0, 0)),
        pl.BlockSpec((_H, _H), lambda i: (0, 0)),
        pl.BlockSpec((1, _H), lambda i: (0, 0)),
        pl.BlockSpec((_H, _H), lambda i: (0, 0)),
        pl.BlockSpec((1, _H), lambda i: (0, 0)),
        pl.BlockSpec((_H, 1), lambda i: (0, 0)),
        pl.BlockSpec((1, 1), lambda i: (0, 0)),
    ],
    out_specs=pl.BlockSpec((_G, 1), lambda i: (0, 0)),
    out_shape=jax.ShapeDtypeStruct((_G, 1), jnp.float32),
    scratch_shapes=[
        pltpu.VMEM((_G, _H), jnp.float32),
        pltpu.VMEM((_G, 1), jnp.float32),
    ],
)


def kernel(x, edge_index, batch, W1, b1, W2, b2, LW1, Lb1, LW2, Lb2):
    src = edge_index[0]
    dst = edge_index[1]
    pad_e = _EPAD - _E
    src_p = jnp.concatenate([src, jnp.zeros((pad_e,), jnp.int32)])
    dump = _N + (jnp.arange(pad_e, dtype=jnp.int32) % (_NPAD - _N))
    dst_p = jnp.concatenate([dst, dump])
    src2d = src_p.reshape(_ROWS // _CH, _CH, 128)
    dst2d = dst_p.reshape(_ROWS // _CH, _CH, 128)
    zf = jnp.zeros((_NPAD,), jnp.float32)
    z16 = jnp.zeros((_NPAD, 16), jnp.float32)
    xpad = jnp.pad(x, ((0, _NPAD - _N), (0, 16 - _IN)))
    w1p = jnp.pad(W1, ((0, 16 - _IN), (0, 0)))

    deg_sc, scat1_sc, scat2_sc = _sc_kernels()
    indeg = deg_sc(dst2d, zf)
    dinv, xs = _prep_tc(indeg.reshape(2, _NPAD, 1), xpad)
    acc1 = scat1_sc(src2d, dst2d, xs, z16)
    tab2 = _l1_tc(acc1, xs, dinv, w1p, b1.reshape(1, _H))
    acc2 = scat2_sc(src2d, dst2d, tab2.reshape(4 * _NPAD, 16), z16)
    batch3 = jnp.pad(batch, (0, _NPAD - _N), constant_values=_G).reshape(
        _GRID, 1, _NB)
    out = _l2_tc(acc2, tab2, dinv, batch3, W2, b2.reshape(1, _H),
                 LW1, Lb1.reshape(1, _H), LW2, Lb2.reshape(1, 1))
    return out


# speedup vs baseline: 1.4805x; 1.0272x over previous
import functools

import jax
import jax.numpy as jnp
from jax import lax
from jax.experimental import pallas as pl
from jax.experimental.pallas import tpu as pltpu
from jax.experimental.pallas import tpu_sc as plsc

_N = 100000
_E = 1600000
_IN = 11
_H = 64
_G = 64

_NB = 2048
_GRID = 49
_NPAD = _NB * _GRID
_EPAD = 1605632
_ROWS = _EPAD // 128
_CH = 8
_M1 = 49
_M2 = 98
_TS = _NPAD // 16

@functools.cache
def _sc_kernels():
    mesh = plsc.VectorSubcoreMesh(
        core_axis_name="c", subcore_axis_name="s", num_cores=2, num_subcores=16)
    params = pltpu.CompilerParams(use_tc_tiling_on_sc=False)
    deg = functools.partial(
        pl.kernel,
        out_type=jax.ShapeDtypeStruct((2, _NPAD), jnp.float32),
        mesh=mesh,
        scratch_types=[
            pltpu.VMEM((_CH, 128), jnp.int32),
            pltpu.VMEM((2, 4, 128), jnp.int32),
            pltpu.VMEM((128,), jnp.float32),
            pltpu.VMEM_SHARED((_NPAD,), jnp.float32),
            pltpu.SemaphoreType.DMA,
            pltpu.SemaphoreType.DMA,
        ],
        compiler_params=params,
    )(_deg_body)
    scat1 = functools.partial(
        pl.kernel,
        out_type=jax.ShapeDtypeStruct((2, _NPAD, 16), jnp.float32),
        mesh=mesh,
        scratch_types=[
            pltpu.VMEM((_CH, 128), jnp.int32),
            pltpu.VMEM((_CH, 128), jnp.int32),
            pltpu.VMEM((_CH, 128), jnp.int32),
            pltpu.VMEM((_CH, 128), jnp.int32),
            pltpu.VMEM((_CH, 128, 16), jnp.float32),
            pltpu.VMEM_SHARED((_NPAD, 16), jnp.float32),
            pltpu.SemaphoreType.DMA,
            pltpu.SemaphoreType.DMA,
            pltpu.SemaphoreType.DMA,
        ],
        compiler_params=params,
    )(_scat1_body)
    scat2 = functools.partial(
        pl.kernel,
        out_type=jax.ShapeDtypeStruct((4, _NPAD, 16), jnp.float32),
        mesh=mesh,
        scratch_types=[
            pltpu.VMEM((_CH, 128), jnp.int32),
            pltpu.VMEM((_CH, 128), jnp.int32),
            pltpu.VMEM((_CH, 128), jnp.int32),
            pltpu.VMEM((_CH, 128), jnp.int32),
            pltpu.VMEM((_CH, 128, 16), jnp.float32),
            pltpu.VMEM_SHARED((_NPAD, 16), jnp.float32),
            pltpu.SemaphoreType.DMA,
            pltpu.SemaphoreType.DMA,
            pltpu.SemaphoreType.DMA,
        ],
        compiler_params=params,
    )(_scat2_body)
    return deg, scat1, scat2


def _pipe_pass(ei_hbm, tab_hbm, acc, ibs, ibd, gb, db, rows,
               semI, semG, semS, base_blk, M, gidx_fn):
    def load_idx(m):
        blk = base_blk + jnp.minimum(m, M - 1)
        pltpu.async_copy(ei_hbm.at[0, blk], ibs, semI)
        pltpu.async_copy(ei_hbm.at[1, blk], ibd, semI)

    def wait_idx():
        pltpu.make_async_copy(ei_hbm.at[0, 0], ibs, semI).wait()
        pltpu.make_async_copy(ei_hbm.at[1, 0], ibd, semI).wait()

    def compute():
        for j in range(_CH):
            for o in range(8):
                sl = pl.ds(o * 16, 16)
                gb[j, sl] = gidx_fn(ibs[j, sl])
                db[j, sl] = ibd[j, sl]

    def fire_g():
        for j in range(_CH):
            pltpu.async_copy(tab_hbm.at[gb.at[j]], rows.at[j], semG)

    def wait_g():
        for j in range(_CH):
            pltpu.make_async_copy(tab_hbm.at[pl.ds(0, 128)], rows.at[j],
                                  semG).wait()

    def fire_s():
        for j in range(_CH):
            pltpu.async_copy(rows.at[j], acc.at[db.at[j]], semS, add=True)

    def wait_s():
        for j in range(_CH):
            pltpu.make_async_copy(tab_hbm.at[pl.ds(0, 128)], rows.at[j],
                                  semS).wait()

    load_idx(0)
    wait_idx()
    compute()
    load_idx(1)
    fire_g()
    wait_g()
    fire_s()

    @pl.loop(1, M)
    def _steps(m):
        wait_idx()
        wait_s()
        compute()
        load_idx(m + 1)
        fire_g()
        wait_g()
        fire_s()

    wait_s()
    wait_idx()


def _deg_body(ei_hbm, zf_hbm, out_hbm, ibd, db, ones_v, acc, semI, semS):
    core = lax.axis_index("c")
    sub = lax.axis_index("s")
    wid = sub * 2 + core
    for o in range(8):
        ones_v[pl.ds(o * 16, 16)] = jnp.ones((16,), jnp.float32)
    pltpu.sync_copy(zf_hbm.at[pl.ds(sub * _TS, _TS)], acc.at[pl.ds(sub * _TS, _TS)])
    plsc.subcore_barrier()
    base_blk = wid * _M1

    def load_idx(m):
        blk = base_blk + jnp.minimum(m, _M1 - 1)
        pltpu.async_copy(ei_hbm.at[1, blk], ibd, semI)

    def wait_idx():
        pltpu.make_async_copy(ei_hbm.at[1, 0], ibd, semI).wait()

    def compute(h, k):
        for j in range(4):
            for o in range(8):
                sl = pl.ds(o * 16, 16)
                db[k, j, sl] = ibd[4 * h + j, sl]

    def fire_s(k):
        for j in range(4):
            pltpu.async_copy(ones_v, acc.at[db.at[k, j]], semS, add=True)

    def wait_s(k):
        for j in range(4):
            pltpu.make_async_copy(zf_hbm.at[pl.ds(0, 128)], ones_v, semS).wait()

    load_idx(0)
    wait_idx()
    compute(0, 0)
    fire_s(0)
    compute(1, 1)
    fire_s(1)
    load_idx(1)

    @pl.loop(0, _M1 - 1)
    def _pairs(p):
        wait_s(0)
        wait_idx()
        compute(0, 0)
        fire_s(0)
        wait_s(1)
        compute(1, 1)
        fire_s(1)
        load_idx(p + 2)

    wait_s(0)
    wait_s(1)
    wait_idx()
    plsc.subcore_barrier()
    pltpu.sync_copy(acc.at[pl.ds(sub * _TS, _TS)],
                    out_hbm.at[core, pl.ds(sub * _TS, _TS)])


def _scat1_body(ei_hbm, tab_hbm, z16_hbm, out_hbm,
                ibs, ibd, gb, db, rows, acc, semI, semG, semS):
    core = lax.axis_index("c")
    sub = lax.axis_index("s")
    wid = sub * 2 + core
    pltpu.sync_copy(z16_hbm.at[pl.ds(sub * _TS, _TS)], acc.at[pl.ds(sub * _TS, _TS)])
    plsc.subcore_barrier()
    _pipe_pass(ei_hbm, tab_hbm, acc, ibs, ibd, gb, db, rows,
               semI, semG, semS, wid * _M1, _M1, lambda v: v)
    plsc.subcore_barrier()
    pltpu.sync_copy(acc.at[pl.ds(sub * _TS, _TS)],
                    out_hbm.at[core, pl.ds(sub * _TS, _TS)])


def _scat2_body(ei_hbm, tab_hbm, z16_hbm, out_hbm,
                ibs, ibd, gb, db, rows, acc, semI, semG, semS):
    core = lax.axis_index("c")
    sub = lax.axis_index("s")
    for cc in range(2):
        c = core * 2 + cc
        pltpu.sync_copy(z16_hbm.at[pl.ds(sub * _TS, _TS)],
                        acc.at[pl.ds(sub * _TS, _TS)])
        plsc.subcore_barrier()
        _pipe_pass(ei_hbm, tab_hbm, acc, ibs, ibd, gb, db, rows,
                   semI, semG, semS, sub * _M2, _M2, lambda v: v * 4 + c)
        plsc.subcore_barrier()
        pltpu.sync_copy(acc.at[pl.ds(sub * _TS, _TS)],
                        out_hbm.at[c, pl.ds(sub * _TS, _TS)])


def _prep_body(indeg_ref, x_ref, dinv_ref, xs_ref):
    i = pl.program_id(0)
    ind = indeg_ref[...]
    s = ind[0] + ind[1]
    row = lax.broadcasted_iota(jnp.int32, (_NB, 1), 0) + i * _NB
    dinv = jnp.where(row < _N, lax.rsqrt(s + 1.0), 0.0)
    dinv_ref[...] = jnp.broadcast_to(dinv, (_NB, 16))
    xs_ref[...] = x_ref[...] * dinv


_prep_tc = pl.pallas_call(
    _prep_body,
    grid=(_GRID,),
    in_specs=[
        pl.BlockSpec((2, _NB, 1), lambda i: (0, i, 0)),
        pl.BlockSpec((_NB, 16), lambda i: (i, 0)),
    ],
    out_specs=[
        pl.BlockSpec((_NB, 16), lambda i: (i, 0)),
        pl.BlockSpec((_NB, 16), lambda i: (i, 0)),
    ],
    out_shape=[
        jax.ShapeDtypeStruct((_NPAD, 16), jnp.float32),
        jax.ShapeDtypeStruct((_NPAD, 16), jnp.float32),
    ],
)


def _l1_body(acc_ref, xs_ref, dinv_ref, w_ref, b_ref, tab_ref):
    a = acc_ref[...]
    t = a[0] + a[1] + xs_ref[...]
    h = jnp.dot(t, w_ref[...], preferred_element_type=jnp.float32)
    dinv = dinv_ref[...][:, 0:1]
    out1 = jnp.maximum(h * dinv + b_ref[...], 0.0)
    tab_ref[...] = out1 * dinv


_l1_tc = pl.pallas_call(
    _l1_body,
    grid=(_GRID,),
    in_specs=[
        pl.BlockSpec((2, _NB, 16), lambda i: (0, i, 0)),
        pl.BlockSpec((_NB, 16), lambda i: (i, 0)),
        pl.BlockSpec((_NB, 16), lambda i: (i, 0)),
        pl.BlockSpec((16, _H), lambda i: (0, 0)),
        pl.BlockSpec((1, _H), lambda i: (0, 0)),
    ],
    out_specs=pl.BlockSpec((_NB, _H), lambda i: (i, 0)),
    out_shape=jax.ShapeDtypeStruct((_NPAD, _H), jnp.float32),
)


def _l2_body(acc_ref, tab_ref, dinv_ref, batch_ref, w2_ref, b2_ref,
             lw1_ref, lb1_ref, lw2_ref, lb2_ref, out_ref, sums, cnts):
    i = pl.program_id(0)

    @pl.when(i == 0)
    def _():
        sums[...] = jnp.zeros((_G, _H), jnp.float32)
        cnts[...] = jnp.zeros((_G, 1), jnp.float32)

    a = acc_ref[...]
    acc = jnp.concatenate([a[0], a[1], a[2], a[3]], axis=-1)
    t = acc + tab_ref[...]
    h = jnp.dot(t, w2_ref[...], preferred_element_type=jnp.float32)
    h2 = jnp.maximum(h * dinv_ref[...][:, 0:1] + b2_ref[...], 0.0)
    b = batch_ref[0]
    io = lax.broadcasted_iota(jnp.int32, (_G, _NB), 0)
    oh = jnp.where(io == b, 1.0, 0.0)
    sums[...] += jnp.dot(oh, h2, preferred_element_type=jnp.float32)
    cnts[...] += jnp.sum(oh, axis=1, keepdims=True)

    @pl.when(i == _GRID - 1)
    def _():
        p = sums[...] / jnp.maximum(cnts[...], 1.0)
        q = jnp.maximum(
            jnp.dot(p, lw1_ref[...], preferred_element_type=jnp.float32)
            + lb1_ref[...], 0.0)
        out_ref[...] = (jnp.dot(q, lw2_ref[...], preferred_element_type=jnp.float32)
                        + lb2_ref[...])


_l2_tc = pl.pallas_call(
    _l2_body,
    grid=(_GRID,),
    in_specs=[
        pl.BlockSpec((4, _NB, 16), lambda i: (0, i, 0)),
        pl.BlockSpec((_NB, _H), lambda i: (i, 0)),
        pl.BlockSpec((_NB, 16), lambda i: (i, 0)),
        pl.BlockSpec((1, 1, _NB), lambda i: (i, 0, 0)),
        pl.BlockSpec((_H, _H), lambda i: (0, 0)),
        pl.BlockSpec((1, _H), lambda i: (0, 0)),
        pl.BlockSpec((_H, _H), lambda i: (0, 0)),
        pl.BlockSpec((1, _H), lambda i: (0, 0)),
        pl.BlockSpec((_H, 1), lambda i: (0, 0)),
        pl.BlockSpec((1, 1), lambda i: (0, 0)),
    ],
    out_specs=pl.BlockSpec((_G, 1), lambda i: (0, 0)),
    out_shape=jax.ShapeDtypeStruct((_G, 1), jnp.float32),
    scratch_shapes=[
        pltpu.VMEM((_G, _H), jnp.float32),
        pltpu.VMEM((_G, 1), jnp.float32),
    ],
)


def kernel(x, edge_index, batch, W1, b1, W2, b2, LW1, Lb1, LW2, Lb2):
    ei4 = jnp.pad(edge_index, ((0, 0), (0, _EPAD - _E)),
                  constant_values=_N).reshape(2, _ROWS // _CH, _CH, 128)
    zf = jnp.zeros((_NPAD,), jnp.float32)
    z16 = jnp.zeros((_NPAD, 16), jnp.float32)
    xpad = jnp.pad(x, ((0, _NPAD - _N), (0, 16 - _IN)))
    w1p = jnp.pad(W1, ((0, 16 - _IN), (0, 0)))

    deg_sc, scat1_sc, scat2_sc = _sc_kernels()
    indeg = deg_sc(ei4, zf)
    dinv, xs = _prep_tc(indeg.reshape(2, _NPAD, 1), xpad)
    acc1 = scat1_sc(ei4, xs, z16)
    tab2 = _l1_tc(acc1, xs, dinv, w1p, b1.reshape(1, _H))
    acc2 = scat2_sc(ei4, tab2.reshape(4 * _NPAD, 16), z16)
    batch3 = jnp.pad(batch, (0, _NPAD - _N), constant_values=_G).reshape(
        _GRID, 1, _NB)
    out = _l2_tc(acc2, tab2, dinv, batch3, W2, b2.reshape(1, _H),
                 LW1, Lb1.reshape(1, _H), LW2, Lb2.reshape(1, 1))
    return out


# speedup vs baseline: 1.4915x; 1.0075x over previous
import functools

import jax
import jax.numpy as jnp
from jax import lax
from jax.experimental import pallas as pl
from jax.experimental.pallas import tpu as pltpu
from jax.experimental.pallas import tpu_sc as plsc

_N = 100000
_E = 1600000
_IN = 11
_H = 64
_G = 64

_NB = 2048
_GRID = 49
_NPAD = _NB * _GRID
_EPAD = 1605632
_ROWS = _EPAD // 128
_CH = 8
_M1 = 49
_M2 = 98
_TS = _NPAD // 16

@functools.cache
def _sc_kernels():
    mesh = plsc.VectorSubcoreMesh(
        core_axis_name="c", subcore_axis_name="s", num_cores=2, num_subcores=16)
    params = pltpu.CompilerParams(use_tc_tiling_on_sc=False, skip_device_barrier=True)
    deg = functools.partial(
        pl.kernel,
        out_type=jax.ShapeDtypeStruct((2, _NPAD), jnp.float32),
        mesh=mesh,
        scratch_types=[
            pltpu.VMEM((_CH, 128), jnp.int32),
            pltpu.VMEM((2, 4, 128), jnp.int32),
            pltpu.VMEM((128,), jnp.float32),
            pltpu.VMEM_SHARED((_NPAD,), jnp.float32),
            pltpu.SemaphoreType.DMA,
            pltpu.SemaphoreType.DMA,
        ],
        compiler_params=params,
    )(_deg_body)
    scat1 = functools.partial(
        pl.kernel,
        out_type=jax.ShapeDtypeStruct((2, _NPAD, 16), jnp.float32),
        mesh=mesh,
        scratch_types=[
            pltpu.VMEM((_CH, 128), jnp.int32),
            pltpu.VMEM((_CH, 128), jnp.int32),
            pltpu.VMEM((_CH, 128), jnp.int32),
            pltpu.VMEM((_CH, 128), jnp.int32),
            pltpu.VMEM((_CH, 128, 16), jnp.float32),
            pltpu.VMEM_SHARED((_NPAD, 16), jnp.float32),
            pltpu.SemaphoreType.DMA,
            pltpu.SemaphoreType.DMA,
            pltpu.SemaphoreType.DMA,
        ],
        compiler_params=params,
    )(_scat1_body)
    scat2 = functools.partial(
        pl.kernel,
        out_type=jax.ShapeDtypeStruct((4, _NPAD, 16), jnp.float32),
        mesh=mesh,
        scratch_types=[
            pltpu.VMEM((_CH, 128), jnp.int32),
            pltpu.VMEM((_CH, 128), jnp.int32),
            pltpu.VMEM((_CH, 128), jnp.int32),
            pltpu.VMEM((_CH, 128), jnp.int32),
            pltpu.VMEM((_CH, 128, 16), jnp.float32),
            pltpu.VMEM_SHARED((_NPAD, 16), jnp.float32),
            pltpu.SemaphoreType.DMA,
            pltpu.SemaphoreType.DMA,
            pltpu.SemaphoreType.DMA,
        ],
        compiler_params=params,
    )(_scat2_body)
    return deg, scat1, scat2


def _pipe_pass(ei_hbm, tab_hbm, acc, ibs, ibd, gb, db, rows,
               semI, semG, semS, base_blk, M, gidx_fn):
    def load_idx(m):
        blk = base_blk + jnp.minimum(m, M - 1)
        pltpu.async_copy(ei_hbm.at[0, blk], ibs, semI)
        pltpu.async_copy(ei_hbm.at[1, blk], ibd, semI)

    def wait_idx():
        pltpu.make_async_copy(ei_hbm.at[0, 0], ibs, semI).wait()
        pltpu.make_async_copy(ei_hbm.at[1, 0], ibd, semI).wait()

    def compute():
        for j in range(_CH):
            for o in range(8):
                sl = pl.ds(o * 16, 16)
                gb[j, sl] = gidx_fn(ibs[j, sl])
                db[j, sl] = ibd[j, sl]

    def fire_g():
        for j in range(_CH):
            pltpu.async_copy(tab_hbm.at[gb.at[j]], rows.at[j], semG)

    def wait_g():
        for j in range(_CH):
            pltpu.make_async_copy(tab_hbm.at[pl.ds(0, 128)], rows.at[j],
                                  semG).wait()

    def fire_s():
        for j in range(_CH):
            pltpu.async_copy(rows.at[j], acc.at[db.at[j]], semS, add=True)

    def wait_s():
        for j in range(_CH):
            pltpu.make_async_copy(tab_hbm.at[pl.ds(0, 128)], rows.at[j],
                                  semS).wait()

    load_idx(0)
    wait_idx()
    compute()
    load_idx(1)
    fire_g()
    wait_g()
    fire_s()

    @pl.loop(1, M)
    def _steps(m):
        wait_idx()
        wait_s()
        compute()
        load_idx(m + 1)
        fire_g()
        wait_g()
        fire_s()

    wait_s()
    wait_idx()


def _deg_body(ei_hbm, zf_hbm, out_hbm, ibd, db, ones_v, acc, semI, semS):
    core = lax.axis_index("c")
    sub = lax.axis_index("s")
    wid = sub * 2 + core
    for o in range(8):
        ones_v[pl.ds(o * 16, 16)] = jnp.ones((16,), jnp.float32)
    pltpu.sync_copy(zf_hbm.at[pl.ds(sub * _TS, _TS)], acc.at[pl.ds(sub * _TS, _TS)])
    plsc.subcore_barrier()
    base_blk = wid * _M1

    def load_idx(m):
        blk = base_blk + jnp.minimum(m, _M1 - 1)
        pltpu.async_copy(ei_hbm.at[1, blk], ibd, semI)

    def wait_idx():
        pltpu.make_async_copy(ei_hbm.at[1, 0], ibd, semI).wait()

    def compute(h, k):
        for j in range(4):
            for o in range(8):
                sl = pl.ds(o * 16, 16)
                db[k, j, sl] = ibd[4 * h + j, sl]

    def fire_s(k):
        for j in range(4):
            pltpu.async_copy(ones_v, acc.at[db.at[k, j]], semS, add=True)

    def wait_s(k):
        for j in range(4):
            pltpu.make_async_copy(zf_hbm.at[pl.ds(0, 128)], ones_v, semS).wait()

    load_idx(0)
    wait_idx()
    compute(0, 0)
    fire_s(0)
    compute(1, 1)
    fire_s(1)
    load_idx(1)

    @pl.loop(0, _M1 - 1)
    def _pairs(p):
        wait_s(0)
        wait_idx()
        compute(0, 0)
        fire_s(0)
        wait_s(1)
        compute(1, 1)
        fire_s(1)
        load_idx(p + 2)

    wait_s(0)
    wait_s(1)
    wait_idx()
    plsc.subcore_barrier()
    pltpu.sync_copy(acc.at[pl.ds(sub * _TS, _TS)],
                    out_hbm.at[core, pl.ds(sub * _TS, _TS)])


def _scat1_body(ei_hbm, tab_hbm, z16_hbm, out_hbm,
                ibs, ibd, gb, db, rows, acc, semI, semG, semS):
    core = lax.axis_index("c")
    sub = lax.axis_index("s")
    wid = sub * 2 + core
    @pl.when(core == 0)
    def _():
        pltpu.sync_copy(tab_hbm.at[pl.ds(sub * _TS, _TS)],
                        acc.at[pl.ds(sub * _TS, _TS)])

    @pl.when(core == 1)
    def _():
        pltpu.sync_copy(z16_hbm.at[pl.ds(sub * _TS, _TS)],
                        acc.at[pl.ds(sub * _TS, _TS)])
    plsc.subcore_barrier()
    _pipe_pass(ei_hbm, tab_hbm, acc, ibs, ibd, gb, db, rows,
               semI, semG, semS, wid * _M1, _M1, lambda v: v)
    plsc.subcore_barrier()
    pltpu.sync_copy(acc.at[pl.ds(sub * _TS, _TS)],
                    out_hbm.at[core, pl.ds(sub * _TS, _TS)])


def _scat2_body(ei_hbm, tab_hbm, z16_hbm, out_hbm,
                ibs, ibd, gb, db, rows, acc, semI, semG, semS):
    core = lax.axis_index("c")
    sub = lax.axis_index("s")
    for cc in range(2):
        c = core * 2 + cc
        pltpu.sync_copy(z16_hbm.at[pl.ds(sub * _TS, _TS)],
                        acc.at[pl.ds(sub * _TS, _TS)])
        plsc.subcore_barrier()
        _pipe_pass(ei_hbm, tab_hbm, acc, ibs, ibd, gb, db, rows,
                   semI, semG, semS, sub * _M2, _M2, lambda v: v * 4 + c)
        plsc.subcore_barrier()
        pltpu.sync_copy(acc.at[pl.ds(sub * _TS, _TS)],
                        out_hbm.at[c, pl.ds(sub * _TS, _TS)])


def _prep_body(indeg_ref, x_ref, dinv_ref, xs_ref):
    i = pl.program_id(0)
    ind = indeg_ref[...]
    s = ind[0] + ind[1]
    row = lax.broadcasted_iota(jnp.int32, (_NB, 1), 0) + i * _NB
    dinv = jnp.where(row < _N, lax.rsqrt(s + 1.0), 0.0)
    dinv_ref[...] = jnp.broadcast_to(dinv, (_NB, 16))
    xs_ref[...] = x_ref[...] * dinv


_prep_tc = pl.pallas_call(
    _prep_body,
    grid=(_GRID,),
    in_specs=[
        pl.BlockSpec((2, _NB, 1), lambda i: (0, i, 0)),
        pl.BlockSpec((_NB, 16), lambda i: (i, 0)),
    ],
    out_specs=[
        pl.BlockSpec((_NB, 16), lambda i: (i, 0)),
        pl.BlockSpec((_NB, 16), lambda i: (i, 0)),
    ],
    out_shape=[
        jax.ShapeDtypeStruct((_NPAD, 16), jnp.float32),
        jax.ShapeDtypeStruct((_NPAD, 16), jnp.float32),
    ],
)


def _l1_body(acc_ref, dinv_ref, w_ref, b_ref, tab_ref):
    a = acc_ref[...]
    t = a[0] + a[1]
    h = jnp.dot(t, w_ref[...], preferred_element_type=jnp.float32)
    dinv = dinv_ref[...][:, 0:1]
    out1 = jnp.maximum(h * dinv + b_ref[...], 0.0)
    tab_ref[...] = out1 * dinv


_l1_tc = pl.pallas_call(
    _l1_body,
    grid=(_GRID,),
    in_specs=[
        pl.BlockSpec((2, _NB, 16), lambda i: (0, i, 0)),
        pl.BlockSpec((_NB, 16), lambda i: (i, 0)),
        pl.BlockSpec((16, _H), lambda i: (0, 0)),
        pl.BlockSpec((1, _H), lambda i: (0, 0)),
    ],
    out_specs=pl.BlockSpec((_NB, _H), lambda i: (i, 0)),
    out_shape=jax.ShapeDtypeStruct((_NPAD, _H), jnp.float32),
)


def _l2_body(acc_ref, tab_ref, dinv_ref, batch_ref, w2_ref, b2_ref,
             lw1_ref, lb1_ref, lw2_ref, lb2_ref, out_ref, sums, cnts):
    i = pl.program_id(0)

    @pl.when(i == 0)
    def _():
        sums[...] = jnp.zeros((_G, _H), jnp.float32)
        cnts[...] = jnp.zeros((_G, 1), jnp.float32)

    a = acc_ref[...]
    acc = jnp.concatenate([a[0], a[1], a[2], a[3]], axis=-1)
    t = acc + tab_ref[...]
    h = jnp.dot(t, w2_ref[...], preferred_element_type=jnp.float32)
    h2 = jnp.maximum(h * dinv_ref[...][:, 0:1] + b2_ref[...], 0.0)
    b = batch_ref[0]
    io = lax.broadcasted_iota(jnp.int32, (_G, _NB), 0)
    oh = jnp.where(io == b, 1.0, 0.0)
    sums[...] += jnp.dot(oh, h2, preferred_element_type=jnp.float32)
    cnts[...] += jnp.sum(oh, axis=1, keepdims=True)

    @pl.when(i == _GRID - 1)
    def _():
        p = sums[...] / jnp.maximum(cnts[...], 1.0)
        q = jnp.maximum(
            jnp.dot(p, lw1_ref[...], preferred_element_type=jnp.float32)
            + lb1_ref[...], 0.0)
        out_ref[...] = (jnp.dot(q, lw2_ref[...], preferred_element_type=jnp.float32)
                        + lb2_ref[...])


_l2_tc = pl.pallas_call(
    _l2_body,
    grid=(_GRID,),
    in_specs=[
        pl.BlockSpec((4, _NB, 16), lambda i: (0, i, 0)),
        pl.BlockSpec((_NB, _H), lambda i: (i, 0)),
        pl.BlockSpec((_NB, 16), lambda i: (i, 0)),
        pl.BlockSpec((1, 1, _NB), lambda i: (i, 0, 0)),
        pl.BlockSpec((_H, _H), lambda i: (0, 0)),
        pl.BlockSpec((1, _H), lambda i: (0, 0)),
        pl.BlockSpec((_H, _H), lambda i: (0, 0)),
        pl.BlockSpec((1, _H), lambda i: (0, 0)),
        pl.BlockSpec((_H, 1), lambda i: (0, 0)),
        pl.BlockSpec((1, 1), lambda i: (0, 0)),
    ],
    out_specs=pl.BlockSpec((_G, 1), lambda i: (0, 0)),
    out_shape=jax.ShapeDtypeStruct((_G, 1), jnp.float32),
    scratch_shapes=[
        pltpu.VMEM((_G, _H), jnp.float32),
        pltpu.VMEM((_G, 1), jnp.float32),
    ],
)


def kernel(x, edge_index, batch, W1, b1, W2, b2, LW1, Lb1, LW2, Lb2):
    ei4 = jnp.pad(edge_index, ((0, 0), (0, _EPAD - _E)),
                  constant_values=_N).reshape(2, _ROWS // _CH, _CH, 128)
    zf = jnp.zeros((_NPAD,), jnp.float32)
    z16 = jnp.zeros((_NPAD, 16), jnp.float32)
    xpad = jnp.pad(x, ((0, _NPAD - _N), (0, 16 - _IN)))
    w1p = jnp.pad(W1, ((0, 16 - _IN), (0, 0)))

    deg_sc, scat1_sc, scat2_sc = _sc_kernels()
    indeg = deg_sc(ei4, zf)
    dinv, xs = _prep_tc(indeg.reshape(2, _NPAD, 1), xpad)
    acc1 = scat1_sc(ei4, xs, z16)
    tab2 = _l1_tc(acc1, dinv, w1p, b1.reshape(1, _H))
    acc2 = scat2_sc(ei4, tab2.reshape(4 * _NPAD, 16), z16)
    batch3 = jnp.pad(batch, (0, _NPAD - _N), constant_values=_G).reshape(
        _GRID, 1, _NB)
    out = _l2_tc(acc2, tab2, dinv, batch3, W2, b2.reshape(1, _H),
                 LW1, Lb1.reshape(1, _H), LW2, Lb2.reshape(1, 1))
    return out


# speedup vs baseline: 1.5777x; 1.0578x over previous
import functools

import jax
import jax.numpy as jnp
from jax import lax
from jax.experimental import pallas as pl
from jax.experimental.pallas import tpu as pltpu
from jax.experimental.pallas import tpu_sc as plsc

_N = 100000
_E = 1600000
_IN = 11
_H = 64
_G = 64

_NB = 2048
_GRID = 49
_NPAD = _NB * _GRID
_EPAD = 1605632
_ROWS = _EPAD // 128
_CH = 8
_M1 = 49
_M2 = 98
_TS = _NPAD // 16

@functools.cache
def _sc_kernels():
    mesh = plsc.VectorSubcoreMesh(
        core_axis_name="c", subcore_axis_name="s", num_cores=2, num_subcores=16)
    params = pltpu.CompilerParams(use_tc_tiling_on_sc=False, skip_device_barrier=True,
                                  needs_layout_passes=False)
    deg = functools.partial(
        pl.kernel,
        out_type=[jax.ShapeDtypeStruct((_NPAD, 16), jnp.float32),
                  jax.ShapeDtypeStruct((_NPAD, 16), jnp.float32)],
        mesh=mesh,
        scratch_types=[
            pltpu.VMEM((_CH, 128), jnp.int32),
            pltpu.VMEM((2, 4, 128), jnp.int32),
            pltpu.VMEM((128,), jnp.float32),
            pltpu.VMEM((_NPAD // 32,), jnp.float32),
            pltpu.VMEM((_NPAD // 32, 16), jnp.float32),
            pltpu.VMEM_SHARED((_NPAD,), jnp.float32),
            pltpu.SemaphoreType.DMA,
            pltpu.SemaphoreType.DMA,
        ],
        compiler_params=params,
    )(_deg_body)
    scat1 = functools.partial(
        pl.kernel,
        out_type=jax.ShapeDtypeStruct((2, _NPAD, 16), jnp.float32),
        mesh=mesh,
        scratch_types=[
            pltpu.VMEM((_CH, 128), jnp.int32),
            pltpu.VMEM((_CH, 128), jnp.int32),
            pltpu.VMEM((_CH, 128), jnp.int32),
            pltpu.VMEM((_CH, 128), jnp.int32),
            pltpu.VMEM((_CH, 128, 16), jnp.float32),
            pltpu.VMEM_SHARED((_NPAD, 16), jnp.float32),
            pltpu.SemaphoreType.DMA,
            pltpu.SemaphoreType.DMA,
            pltpu.SemaphoreType.DMA,
        ],
        compiler_params=params,
    )(_scat1_body)
    scat2 = functools.partial(
        pl.kernel,
        out_type=jax.ShapeDtypeStruct((4, _NPAD, 16), jnp.float32),
        mesh=mesh,
        scratch_types=[
            pltpu.VMEM((_CH, 128), jnp.int32),
            pltpu.VMEM((_CH, 128), jnp.int32),
            pltpu.VMEM((_CH, 128), jnp.int32),
            pltpu.VMEM((_CH, 128), jnp.int32),
            pltpu.VMEM((_CH, 128, 16), jnp.float32),
            pltpu.VMEM_SHARED((_NPAD, 16), jnp.float32),
            pltpu.SemaphoreType.DMA,
            pltpu.SemaphoreType.DMA,
            pltpu.SemaphoreType.DMA,
        ],
        compiler_params=params,
    )(_scat2_body)
    return deg, scat1, scat2


def _pipe_pass(ei_hbm, tab_hbm, acc, ibs, ibd, gb, db, rows,
               semI, semG, semS, base_blk, M, gidx_fn):
    def load_idx(m):
        blk = base_blk + jnp.minimum(m, M - 1)
        pltpu.async_copy(ei_hbm.at[0, blk], ibs, semI)
        pltpu.async_copy(ei_hbm.at[1, blk], ibd, semI)

    def wait_idx():
        pltpu.make_async_copy(ei_hbm.at[0, 0], ibs, semI).wait()
        pltpu.make_async_copy(ei_hbm.at[1, 0], ibd, semI).wait()

    def compute():
        for j in range(_CH):
            for o in range(8):
                sl = pl.ds(o * 16, 16)
                gb[j, sl] = gidx_fn(ibs[j, sl])
                db[j, sl] = ibd[j, sl]

    def fire_g():
        for j in range(_CH):
            pltpu.async_copy(tab_hbm.at[gb.at[j]], rows.at[j], semG)

    def wait_g():
        for j in range(_CH):
            pltpu.make_async_copy(tab_hbm.at[pl.ds(0, 128)], rows.at[j],
                                  semG).wait()

    def fire_s():
        for j in range(_CH):
            pltpu.async_copy(rows.at[j], acc.at[db.at[j]], semS, add=True)

    def wait_s():
        for j in range(_CH):
            pltpu.make_async_copy(tab_hbm.at[pl.ds(0, 128)], rows.at[j],
                                  semS).wait()

    load_idx(0)
    wait_idx()
    compute()
    load_idx(1)
    fire_g()
    wait_g()
    fire_s()

    @pl.loop(1, M)
    def _steps(m):
        wait_idx()
        wait_s()
        compute()
        load_idx(m + 1)
        fire_g()
        wait_g()
        fire_s()

    wait_s()
    wait_idx()


_NS = _NPAD // 32


def _deg_body(ei_hbm, zf_hbm, x_hbm, dinv_hbm, xs_hbm,
              ibd, db, ones_v, dv, xv, acc, semI, semS):
    core = lax.axis_index("c")
    sub = lax.axis_index("s")
    for o in range(8):
        ones_v[pl.ds(o * 16, 16)] = jnp.ones((16,), jnp.float32)
    pltpu.sync_copy(zf_hbm.at[pl.ds(sub * _TS, _TS)], acc.at[pl.ds(sub * _TS, _TS)])
    plsc.subcore_barrier()
    base_blk = sub * (2 * _M1)
    M = 2 * _M1

    def load_idx(m):
        blk = base_blk + jnp.minimum(m, M - 1)
        pltpu.async_copy(ei_hbm.at[1, blk], ibd, semI)

    def wait_idx():
        pltpu.make_async_copy(ei_hbm.at[1, 0], ibd, semI).wait()

    def compute(h, k):
        for j in range(4):
            for o in range(8):
                sl = pl.ds(o * 16, 16)
                db[k, j, sl] = ibd[4 * h + j, sl]

    def fire_s(k):
        for j in range(4):
            pltpu.async_copy(ones_v, acc.at[db.at[k, j]], semS, add=True)

    def wait_s(k):
        for j in range(4):
            pltpu.make_async_copy(zf_hbm.at[pl.ds(0, 128)], ones_v, semS).wait()

    load_idx(0)
    wait_idx()
    compute(0, 0)
    fire_s(0)
    compute(1, 1)
    fire_s(1)
    load_idx(1)

    @pl.loop(0, M - 1)
    def _pairs(p):
        wait_s(0)
        wait_idx()
        compute(0, 0)
        fire_s(0)
        wait_s(1)
        compute(1, 1)
        fire_s(1)
        load_idx(p + 2)

    wait_s(0)
    wait_s(1)
    wait_idx()
    plsc.subcore_barrier()

    wid = core * 16 + sub
    node0 = wid * _NS
    pltpu.sync_copy(acc.at[pl.ds(node0, _NS)], dv)
    pltpu.sync_copy(x_hbm.at[pl.ds(node0, _NS)], xv)

    @pl.loop(0, _NS // 16)
    def _rsqrt(r):
        d = dv[pl.ds(r * 16, 16)] + 1.0
        i = plsc.bitcast(d, jnp.int32)
        y = plsc.bitcast(0x5F3759DF - lax.shift_right_logical(i, 1), jnp.float32)
        for _ in range(3):
            y = y * (1.5 - 0.5 * d * y * y)
        dv[pl.ds(r * 16, 16)] = y

    @pl.loop(0, _NS // 16)
    def _scale(r):
        for k in range(16):
            n = r * 16 + k
            dd = plsc.load_gather(dv, [jnp.full((16,), n, jnp.int32)])
            xv[n, :] = xv[n, :] * dd

    pltpu.sync_copy(xv, xs_hbm.at[pl.ds(node0, _NS)])

    @pl.loop(0, _NS // 16)
    def _rep(r):
        for k in range(16):
            n = r * 16 + k
            xv[n, :] = plsc.load_gather(dv, [jnp.full((16,), n, jnp.int32)])

    pltpu.sync_copy(xv, dinv_hbm.at[pl.ds(node0, _NS)])


def _scat1_body(ei_hbm, tab_hbm, z16_hbm, out_hbm,
                ibs, ibd, gb, db, rows, acc, semI, semG, semS):
    core = lax.axis_index("c")
    sub = lax.axis_index("s")
    wid = sub * 2 + core
    @pl.when(core == 0)
    def _():
        pltpu.sync_copy(tab_hbm.at[pl.ds(sub * _TS, _TS)],
                        acc.at[pl.ds(sub * _TS, _TS)])

    @pl.when(core == 1)
    def _():
        pltpu.sync_copy(z16_hbm.at[pl.ds(sub * _TS, _TS)],
                        acc.at[pl.ds(sub * _TS, _TS)])
    plsc.subcore_barrier()
    _pipe_pass(ei_hbm, tab_hbm, acc, ibs, ibd, gb, db, rows,
               semI, semG, semS, wid * _M1, _M1, lambda v: v)
    plsc.subcore_barrier()
    pltpu.sync_copy(acc.at[pl.ds(sub * _TS, _TS)],
                    out_hbm.at[core, pl.ds(sub * _TS, _TS)])


def _scat2_body(ei_hbm, tab_hbm, z16_hbm, out_hbm,
                ibs, ibd, gb, db, rows, acc, semI, semG, semS):
    core = lax.axis_index("c")
    sub = lax.axis_index("s")
    for cc in range(2):
        c = core * 2 + cc
        pltpu.sync_copy(z16_hbm.at[pl.ds(sub * _TS, _TS)],
                        acc.at[pl.ds(sub * _TS, _TS)])
        plsc.subcore_barrier()
        _pipe_pass(ei_hbm, tab_hbm, acc, ibs, ibd, gb, db, rows,
                   semI, semG, semS, sub * _M2, _M2, lambda v: v * 4 + c)
        plsc.subcore_barrier()
        pltpu.sync_copy(acc.at[pl.ds(sub * _TS, _TS)],
                        out_hbm.at[c, pl.ds(sub * _TS, _TS)])


def _prep_body(indeg_ref, x_ref, dinv_ref, xs_ref):
    i = pl.program_id(0)
    ind = indeg_ref[...]
    s = ind[0] + ind[1]
    row = lax.broadcasted_iota(jnp.int32, (_NB, 1), 0) + i * _NB
    dinv = jnp.where(row < _N, lax.rsqrt(s + 1.0), 0.0)
    dinv_ref[...] = jnp.broadcast_to(dinv, (_NB, 16))
    xs_ref[...] = x_ref[...] * dinv


_prep_tc = pl.pallas_call(
    _prep_body,
    grid=(_GRID,),
    in_specs=[
        pl.BlockSpec((2, _NB, 1), lambda i: (0, i, 0)),
        pl.BlockSpec((_NB, 16), lambda i: (i, 0)),
    ],
    out_specs=[
        pl.BlockSpec((_NB, 16), lambda i: (i, 0)),
        pl.BlockSpec((_NB, 16), lambda i: (i, 0)),
    ],
    out_shape=[
        jax.ShapeDtypeStruct((_NPAD, 16), jnp.float32),
        jax.ShapeDtypeStruct((_NPAD, 16), jnp.float32),
    ],
)


def _l1_body(acc_ref, dinv_ref, w_ref, b_ref, tab_ref):
    a = acc_ref[...]
    t = a[0] + a[1]
    h = jnp.dot(t, w_ref[...], preferred_element_type=jnp.float32)
    dinv = dinv_ref[...][:, 0:1]
    out1 = jnp.maximum(h * dinv + b_ref[...], 0.0)
    tab_ref[...] = out1 * dinv


_l1_tc = pl.pallas_call(
    _l1_body,
    grid=(_GRID,),
    in_specs=[
        pl.BlockSpec((2, _NB, 16), lambda i: (0, i, 0)),
        pl.BlockSpec((_NB, 16), lambda i: (i, 0)),
        pl.BlockSpec((16, _H), lambda i: (0, 0)),
        pl.BlockSpec((1, _H), lambda i: (0, 0)),
    ],
    out_specs=pl.BlockSpec((_NB, _H), lambda i: (i, 0)),
    out_shape=jax.ShapeDtypeStruct((_NPAD, _H), jnp.float32),
)


def _l2_body(acc_ref, tab_ref, dinv_ref, batch_ref, w2_ref, b2_ref,
             lw1_ref, lb1_ref, lw2_ref, lb2_ref, out_ref, sums, cnts):
    i = pl.program_id(0)

    @pl.when(i == 0)
    def _():
        sums[...] = jnp.zeros((_G, _H), jnp.float32)
        cnts[...] = jnp.zeros((_G, 1), jnp.float32)

    a = acc_ref[...]
    acc = jnp.concatenate([a[0], a[1], a[2], a[3]], axis=-1)
    t = acc + tab_ref[...]
    h = jnp.dot(t, w2_ref[...], preferred_element_type=jnp.float32)
    h2 = jnp.maximum(h * dinv_ref[...][:, 0:1] + b2_ref[...], 0.0)
    b = batch_ref[0]
    io = lax.broadcasted_iota(jnp.int32, (_G, _NB), 0)
    oh = jnp.where(io == b, 1.0, 0.0)
    sums[...] += jnp.dot(oh, h2, preferred_element_type=jnp.float32)
    cnts[...] += jnp.sum(oh, axis=1, keepdims=True)

    @pl.when(i == _GRID - 1)
    def _():
        p = sums[...] / jnp.maximum(cnts[...], 1.0)
        q = jnp.maximum(
            jnp.dot(p, lw1_ref[...], preferred_element_type=jnp.float32)
            + lb1_ref[...], 0.0)
        out_ref[...] = (jnp.dot(q, lw2_ref[...], preferred_element_type=jnp.float32)
                        + lb2_ref[...])


_l2_tc = pl.pallas_call(
    _l2_body,
    grid=(_GRID,),
    in_specs=[
        pl.BlockSpec((4, _NB, 16), lambda i: (0, i, 0)),
        pl.BlockSpec((_NB, _H), lambda i: (i, 0)),
        pl.BlockSpec((_NB, 16), lambda i: (i, 0)),
        pl.BlockSpec((1, 1, _NB), lambda i: (i, 0, 0)),
        pl.BlockSpec((_H, _H), lambda i: (0, 0)),
        pl.BlockSpec((1, _H), lambda i: (0, 0)),
        pl.BlockSpec((_H, _H), lambda i: (0, 0)),
        pl.BlockSpec((1, _H), lambda i: (0, 0)),
        pl.BlockSpec((_H, 1), lambda i: (0, 0)),
        pl.BlockSpec((1, 1), lambda i: (0, 0)),
    ],
    out_specs=pl.BlockSpec((_G, 1), lambda i: (0, 0)),
    out_shape=jax.ShapeDtypeStruct((_G, 1), jnp.float32),
    scratch_shapes=[
        pltpu.VMEM((_G, _H), jnp.float32),
        pltpu.VMEM((_G, 1), jnp.float32),
    ],
)


def kernel(x, edge_index, batch, W1, b1, W2, b2, LW1, Lb1, LW2, Lb2):
    ei4 = jnp.pad(edge_index, ((0, 0), (0, _EPAD - _E)),
                  constant_values=_N).reshape(2, _ROWS // _CH, _CH, 128)
    zf = jnp.zeros((_NPAD,), jnp.float32)
    z16 = jnp.zeros((_NPAD, 16), jnp.float32)
    xpad = jnp.pad(x, ((0, _NPAD - _N), (0, 16 - _IN)))
    w1p = jnp.pad(W1, ((0, 16 - _IN), (0, 0)))

    deg_sc, scat1_sc, scat2_sc = _sc_kernels()
    dinv, xs = deg_sc(ei4, zf, xpad)
    acc1 = scat1_sc(ei4, xs, z16)
    tab2 = _l1_tc(acc1, dinv, w1p, b1.reshape(1, _H))
    acc2 = scat2_sc(ei4, tab2.reshape(4 * _NPAD, 16), z16)
    batch3 = jnp.pad(batch, (0, _NPAD - _N), constant_values=_G).reshape(
        _GRID, 1, _NB)
    out = _l2_tc(acc2, tab2, dinv, batch3, W2, b2.reshape(1, _H),
                 LW1, Lb1.reshape(1, _H), LW2, Lb2.reshape(1, 1))
    return out
